# baseline (speedup 1.0000x reference)
import jax
import jax.numpy as jnp
from jax.experimental import pallas as pl
from jax.experimental.pallas import tpu as pltpu

B, L, D, K = 8, 1024, 256, 8192
BM = 512


def _norm_slots_kernel(slots_ref, sn_ref):
    s = slots_ref[...]
    n = jnp.sqrt(jnp.sum(s * s, axis=-1, keepdims=True)) + 1e-8
    sn_ref[...] = (s / n).astype(jnp.bfloat16)


def _cosine_kernel(z_ref, sn_ref, out_ref):
    zb = z_ref[...]
    zn = jnp.sqrt(jnp.sum(zb * zb, axis=-1, keepdims=True)) + 1e-8
    zb = (zb / zn).astype(jnp.bfloat16)
    out_ref[...] = jax.lax.dot_general(
        zb, sn_ref[...],
        dimension_numbers=(((1,), (1,)), ((), ())),
        preferred_element_type=jnp.float32,
    )


def kernel(z, slots):
    M = B * L
    z2 = z.reshape(M, D)
    slots_n = pl.pallas_call(
        _norm_slots_kernel,
        out_shape=jax.ShapeDtypeStruct((K, D), jnp.bfloat16),
    )(slots)
    out = pl.pallas_call(
        _cosine_kernel,
        grid=(M // BM,),
        in_specs=[
            pl.BlockSpec((BM, D), lambda i: (i, 0)),
            pl.BlockSpec((K, D), lambda i: (0, 0)),
        ],
        out_specs=pl.BlockSpec((BM, K), lambda i: (i, 0)),
        out_shape=jax.ShapeDtypeStruct((M, K), jnp.float32),
        compiler_params=pltpu.CompilerParams(
            dimension_semantics=("parallel",),
        ),
    )(z2, slots_n)
    return out.reshape(B, L, K)


# speedup vs baseline: 1.0502x; 1.0502x over previous
import jax
import jax.numpy as jnp
from jax.experimental import pallas as pl
from jax.experimental.pallas import tpu as pltpu

B, L, D, K = 8, 1024, 256, 8192
BM = 256


def _cosine_kernel(z_ref, slots_ref, out_ref, sn_ref):
    @pl.when(pl.program_id(0) == 0)
    def _():
        s = slots_ref[...]
        n = jnp.sqrt(jnp.sum(s * s, axis=-1, keepdims=True)) + 1e-8
        sn_ref[...] = (s / n).astype(jnp.bfloat16)

    zb = z_ref[...]
    zn = jnp.sqrt(jnp.sum(zb * zb, axis=-1, keepdims=True)) + 1e-8
    zb = (zb / zn).astype(jnp.bfloat16)
    out_ref[...] = jax.lax.dot_general(
        zb, sn_ref[...],
        dimension_numbers=(((1,), (1,)), ((), ())),
        preferred_element_type=jnp.float32,
    )


def kernel(z, slots):
    M = B * L
    z2 = z.reshape(M, D)
    out = pl.pallas_call(
        _cosine_kernel,
        grid=(M // BM,),
        in_specs=[
            pl.BlockSpec((BM, D), lambda i: (i, 0)),
            pl.BlockSpec((K, D), lambda i: (0, 0)),
        ],
        out_specs=pl.BlockSpec((BM, K), lambda i: (i, 0)),
        out_shape=jax.ShapeDtypeStruct((M, K), jnp.float32),
        scratch_shapes=[pltpu.VMEM((K, D), jnp.bfloat16)],
    )(z2, slots)
    return out.reshape(B, L, K)
